# Initial kernel scaffold; baseline (speedup 1.0000x reference)
import jax
import jax.numpy as jnp
from jax.experimental import pallas as pl


def kernel(input, table):
    raise NotImplementedError("write your pallas kernel here")


# speedup vs baseline: 2.9226x; 2.9226x over previous
import functools

import jax
import jax.numpy as jnp
from jax import lax
from jax.experimental import pallas as pl
from jax.experimental.pallas import tpu as pltpu
from jax.experimental.pallas import tpu_sc as plsc

T = 200
B = 4096
V = 100000
D = 64
DP = 128
DO = D + 2

NC = 2
NS = 16
NW = NC * NS

ROWS = T * B
ROWS_PER_W = ROWS // NW
SUB = 128
CHUNK = 256
JJ = CHUNK // SUB
CHUNKS = ROWS_PER_W // CHUNK
TPER = B // CHUNK

_mesh = plsc.VectorSubcoreMesh(
    core_axis_name="c", subcore_axis_name="s", num_cores=NC, num_subcores=NS
)


@functools.partial(
    pl.kernel,
    out_type=jax.ShapeDtypeStruct((ROWS, DO), jnp.float32),
    mesh=_mesh,
    scratch_types=[
        pltpu.VMEM((JJ, SUB), jnp.int32),
        pltpu.VMEM((CHUNK, DP), jnp.float32),
        pltpu.VMEM((CHUNK, DO), jnp.float32),
        pltpu.VMEM((512,), jnp.float32),
        pltpu.SemaphoreType.DMA,
    ],
)
def _encode(idx_hbm, table_hbm, tf_hbm, out_hbm, idx_v, emb_v, stage_v, tf_v, sem):
    wid = lax.axis_index("s") * NC + lax.axis_index("c")
    pltpu.sync_copy(tf_hbm, tf_v)

    lane = lax.iota(jnp.int32, 16)

    row_base0 = wid * ROWS_PER_W
    idx_row0 = wid * (ROWS_PER_W // SUB)

    @pl.loop(0, CHUNKS)
    def chunk_loop(c):
        base = row_base0 + c * CHUNK
        pltpu.sync_copy(idx_hbm.at[pl.ds(idx_row0 + c * JJ, JJ)], idx_v)
        copies = [
            pltpu.async_copy(
                table_hbm.at[idx_v.at[j]],
                emb_v.at[pl.ds(j * SUB, SUB)],
                sem,
            )
            for j in range(JJ)
        ]

        @pl.when(jnp.logical_or(c == 0, (base & (B - 1)) == 0))
        def fill_time():
            t = jnp.right_shift(base, 12)
            tv = tf_v[pl.ds(2 * t, 16)]
            pat = jnp.where(lane == 14, tv[0], tv[1])

            @pl.loop(0, CHUNK, unroll=8)
            def fill(r):
                stage_v[r, pl.ds(DO - 16, 16)] = pat

        for cp in copies:
            cp.wait()

        @pl.loop(0, CHUNK, unroll=8)
        def assemble(r):
            for k in range(D // 16):
                stage_v[r, pl.ds(k * 16, 16)] = emb_v[r, pl.ds(k * 16, 16)]

        pltpu.sync_copy(stage_v, out_hbm.at[pl.ds(base, CHUNK)])


def kernel(input, table):
    idx = input[:, :, 0].astype(jnp.int32).reshape(ROWS // SUB, SUB)
    tablep = jnp.pad(table, ((0, 0), (0, DP - D)))
    t = jnp.arange(T, dtype=jnp.float32)
    tf = jnp.stack([jnp.log(t + 1.0), jnp.exp(t / 1000.0) - 1.0], axis=-1)
    tf = jnp.pad(tf.reshape(-1), (0, 512 - 2 * T))
    out = _encode(idx, tablep, tf)
    return out.reshape(T, B, DO)


# speedup vs baseline: 3.9697x; 1.3583x over previous
import functools

import jax
import jax.numpy as jnp
from jax import lax
from jax.experimental import pallas as pl
from jax.experimental.pallas import tpu as pltpu
from jax.experimental.pallas import tpu_sc as plsc

T = 200
B = 4096
V = 100000
D = 64
DP = 128
DO = D + 2

NC = 2
NS = 16
NW = NC * NS

ROWS = T * B
ROWS_PER_W = ROWS // NW
SUB = 128
CHUNK = 128
CHUNKS = ROWS_PER_W // CHUNK

_mesh = plsc.VectorSubcoreMesh(
    core_axis_name="c", subcore_axis_name="s", num_cores=NC, num_subcores=NS
)


def _pad_table_kernel(table_ref, out_ref):
    out_ref[:, :D] = table_ref[...]
    out_ref[:, D:] = jnp.zeros_like(out_ref[:, D:])


_PAD_ROWS = 1000


@jax.jit
def _pad_table(table):
    return pl.pallas_call(
        _pad_table_kernel,
        grid=(V // _PAD_ROWS,),
        in_specs=[pl.BlockSpec((_PAD_ROWS, D), lambda i: (i, 0))],
        out_specs=pl.BlockSpec((_PAD_ROWS, DP), lambda i: (i, 0)),
        out_shape=jax.ShapeDtypeStruct((V, DP), jnp.float32),
    )(table)


@functools.partial(
    pl.kernel,
    out_type=jax.ShapeDtypeStruct((ROWS, DO), jnp.float32),
    mesh=_mesh,
    scratch_types=[
        pltpu.VMEM((CHUNKS, SUB), jnp.int32),
        pltpu.VMEM((2, CHUNK, DP), jnp.float32),
        pltpu.VMEM((2, CHUNK, DO), jnp.float32),
        pltpu.VMEM((512,), jnp.float32),
        pltpu.SemaphoreType.DMA,
        pltpu.SemaphoreType.DMA,
        pltpu.SemaphoreType.DMA,
        pltpu.SemaphoreType.DMA,
    ],
)
def _encode(
    idx_hbm, table_hbm, tf_hbm, out_hbm,
    idx_v, emb_v, stage_v, tf_v,
    gsem0, gsem1, wsem0, wsem1,
):
    wid = lax.axis_index("s") * NC + lax.axis_index("c")
    pltpu.sync_copy(tf_hbm, tf_v)
    pltpu.sync_copy(idx_hbm.at[pl.ds(wid * CHUNKS, CHUNKS)], idx_v)

    lane = lax.iota(jnp.int32, 16)
    row_base0 = wid * ROWS_PER_W

    def gsem(b):
        return gsem0 if b == 0 else gsem1

    def wsem(b):
        return wsem0 if b == 0 else wsem1

    def start_gather(c, b):
        pltpu.async_copy(table_hbm.at[idx_v.at[c]], emb_v.at[b], gsem(b))

    def finish_chunk(c, b):
        base = row_base0 + c * CHUNK
        pltpu.make_async_copy(
            table_hbm.at[idx_v.at[c]], emb_v.at[b], gsem(b)
        ).wait()

        @pl.when(jnp.logical_or(c < 2, (base & (B - 1)) < 2 * CHUNK))
        def fill_time():
            t = jnp.right_shift(base, 12)
            tv = tf_v[pl.ds(2 * t, 16)]
            pat = jnp.where(lane == 14, tv[0], tv[1])

            @pl.loop(0, CHUNK, unroll=8)
            def fill(r):
                stage_v[b, r, pl.ds(DO - 16, 16)] = pat

        @pl.loop(0, CHUNK, unroll=8)
        def assemble(r):
            for k in range(D // 16):
                stage_v[b, r, pl.ds(k * 16, 16)] = emb_v[b, r, pl.ds(k * 16, 16)]

        pltpu.async_copy(
            stage_v.at[b], out_hbm.at[pl.ds(base, CHUNK)], wsem(b)
        )

    def wait_writeback(c, b):
        pltpu.make_async_copy(
            stage_v.at[b], out_hbm.at[pl.ds(row_base0 + c * CHUNK, CHUNK)], wsem(b)
        ).wait()

    start_gather(0, 0)

    @pl.loop(1, CHUNKS)
    def chunk_loop(c):
        b = (c & 1) == 1

        @pl.when(c >= 2)
        def wait_stage():
            @pl.when(b)
            def w1():
                wait_writeback(c - 2, 1)

            @pl.when(jnp.logical_not(b))
            def w0():
                wait_writeback(c - 2, 0)

        @pl.when(b)
        def g1():
            start_gather(c, 1)
            finish_chunk(c - 1, 0)

        @pl.when(jnp.logical_not(b))
        def g0():
            start_gather(c, 0)
            finish_chunk(c - 1, 1)

    last = CHUNKS - 1
    lastb = last & 1
    finish_chunk(last, lastb)
    wait_writeback(last - 1, 1 - lastb)
    wait_writeback(last, lastb)


def kernel(input, table):
    idx = input[:, :, 0].astype(jnp.int32).reshape(ROWS // SUB, SUB)
    tablep = _pad_table(table)
    t = jnp.arange(T, dtype=jnp.float32)
    tf = jnp.stack([jnp.log(t + 1.0), jnp.exp(t / 1000.0) - 1.0], axis=-1)
    tf = jnp.pad(tf.reshape(-1), (0, 512 - 2 * T))
    out = _encode(idx, tablep, tf)
    return out.reshape(T, B, DO)
